# baseline (speedup 1.0000x reference)
import functools

import jax
import jax.numpy as jnp
from jax import lax
from jax.experimental import pallas as pl
from jax.experimental.pallas import tpu as pltpu
from jax.experimental.pallas import tpu_sc as plsc

N = 10000
E = 320000
H = 128
G = 64
EPS = 1e-5

NC, NS = 2, 16
NW = NC * NS
EPW = E // NW
CK = 80
NCH = 128
EPT = NCH * CK
ROWS_PT = 640
DEG_EPT = E // NS
NPAD16 = 640

RB = 2000
GRID = N // RB


NROWS = NPAD16 * 16


def _deg_body(dst_hbm, out_hbm, didx, ones_rows, zbuf, shared):
    cid = lax.axis_index("c")
    sid = lax.axis_index("s")
    wid = sid * NC + cid
    ones16 = jnp.ones((16,), jnp.float32)
    zeros16 = jnp.zeros((16,), jnp.float32)

    def fill(r, c):
        ones_rows[r] = ones16
        zbuf[r] = zeros16
        return c
    lax.fori_loop(0, CK, fill, 0)
    for r in range(640 // CK):
        pltpu.sync_copy(zbuf, shared.at[pl.ds(sid * 640 + r * CK, CK)])
    plsc.subcore_barrier()

    pltpu.sync_copy(dst_hbm.at[wid], didx)

    def body(j, c):
        pltpu.sync_copy(ones_rows, shared.at[didx.at[j]], add=True)
        return c
    lax.fori_loop(0, NCH, body, 0)

    plsc.subcore_barrier()
    pltpu.sync_copy(shared.at[pl.ds(sid * 640, 640)],
                    out_hbm.at[cid, pl.ds(sid * 640, 640)])


@functools.cache
def _get_deg_call():
    return pl.kernel(
        _deg_body,
        out_type=jax.ShapeDtypeStruct((NC, NROWS, 16), jnp.float32),
        mesh=plsc.VectorSubcoreMesh(core_axis_name="c", subcore_axis_name="s",
                                    num_cores=NC, num_subcores=NS),
        scratch_types=[
            pltpu.VMEM((NCH, CK), jnp.int32),
            pltpu.VMEM((CK, 16), jnp.float32),
            pltpu.VMEM((CK, 16), jnp.float32),
            pltpu.VMEM_SHARED((NROWS, 16), jnp.float32),
        ],
    )


def _deg_combine_body(p_ref, o_ref):
    o_ref[...] = (jnp.sum(p_ref[0], axis=1, keepdims=True)
                  + jnp.sum(p_ref[1], axis=1, keepdims=True)) / 16.0 + 1.0


def _deg_combine(p):
    return pl.pallas_call(
        _deg_combine_body,
        out_shape=jax.ShapeDtypeStruct((NROWS, 1), jnp.float32),
    )(p)


def _edge_body(xs_hbm, src_hbm, dst_hbm, out_hbm, sidx, didx, rows0,
               shared, sem0):
    cid = lax.axis_index("c")
    sid = lax.axis_index("s")
    wid = sid * NC + cid
    zeros16 = jnp.zeros((16,), jnp.float32)

    def zr(r, c):
        def zc(cc, c2):
            rows0[r, pl.ds(cc * 16, 16)] = zeros16
            return c2
        return lax.fori_loop(0, H // 16, zc, c)
    lax.fori_loop(0, CK, zr, 0)
    for r in range(ROWS_PT // CK):
        pltpu.sync_copy(rows0, shared.at[pl.ds(sid * ROWS_PT + r * CK, CK)])
    plsc.subcore_barrier()

    pltpu.sync_copy(src_hbm.at[wid], sidx)
    pltpu.sync_copy(dst_hbm.at[wid], didx)

    def body(j, c):
        pltpu.async_copy(xs_hbm.at[sidx.at[j]], rows0, sem0).wait()
        pltpu.sync_copy(rows0, shared.at[didx.at[j]], add=True)
        return c
    lax.fori_loop(0, NCH, body, 0)

    plsc.subcore_barrier()
    pltpu.sync_copy(shared.at[pl.ds(sid * ROWS_PT, ROWS_PT)],
                    out_hbm.at[cid, pl.ds(sid * ROWS_PT, ROWS_PT)])


@functools.cache
def _get_edge_call():
    return pl.kernel(
        _edge_body,
        out_type=jax.ShapeDtypeStruct((NC, NROWS, H), jnp.float32),
        mesh=plsc.VectorSubcoreMesh(core_axis_name="c", subcore_axis_name="s",
                                    num_cores=NC, num_subcores=NS),
        scratch_types=[
            pltpu.VMEM((NCH, CK), jnp.int32),
            pltpu.VMEM((NCH, CK), jnp.int32),
            pltpu.VMEM((CK, H), jnp.float32),
            pltpu.VMEM_SHARED((NROWS, H), jnp.float32),
            pltpu.SemaphoreType.DMA,
        ],
    )


def _mm_scale_body(x_ref, w_ref, deg_ref, o_ref):
    dinv = 1.0 / jnp.sqrt(deg_ref[...])
    o_ref[...] = jnp.dot(x_ref[...], w_ref[...],
                         preferred_element_type=jnp.float32) * dinv


def _mm_scale(x, w, deg):
    return pl.pallas_call(
        _mm_scale_body,
        grid=(GRID,),
        in_specs=[pl.BlockSpec((RB, H), lambda i: (i, 0)),
                  pl.BlockSpec((H, H), lambda i: (0, 0)),
                  pl.BlockSpec((RB, 1), lambda i: (i, 0))],
        out_specs=pl.BlockSpec((RB, H), lambda i: (i, 0)),
        out_shape=jax.ShapeDtypeStruct((NROWS, H), jnp.float32),
    )(x, w, deg)


def _stats_body(p_ref, xs_ref, deg_ref, b_ref, t_ref, s_ref):
    i = pl.program_id(0)
    dinv = 1.0 / jnp.sqrt(deg_ref[...])
    t = (p_ref[0] + p_ref[1] + xs_ref[...]) * dinv + b_ref[...]
    t_ref[...] = t

    @pl.when(i == 0)
    def _():
        s_ref[...] = jnp.zeros_like(s_ref)

    s_ref[0:1, :] += jnp.sum(t, axis=0, keepdims=True)
    s_ref[1:2, :] += jnp.sum(t * t, axis=0, keepdims=True)


def _stats(p, xs, deg, b):
    return pl.pallas_call(
        _stats_body,
        grid=(GRID,),
        in_specs=[pl.BlockSpec((NC, RB, H), lambda i: (0, i, 0)),
                  pl.BlockSpec((RB, H), lambda i: (i, 0)),
                  pl.BlockSpec((RB, 1), lambda i: (i, 0)),
                  pl.BlockSpec((1, H), lambda i: (0, 0))],
        out_specs=[pl.BlockSpec((RB, H), lambda i: (i, 0)),
                   pl.BlockSpec((2, H), lambda i: (0, 0))],
        out_shape=[jax.ShapeDtypeStruct((N, H), jnp.float32),
                   jax.ShapeDtypeStruct((2, H), jnp.float32)],
    )(p, xs, deg, b)


def _bn_mm_body(t_ref, s_ref, g_ref, be_ref, w_ref, deg_ref, o_ref):
    mu = s_ref[0:1, :] / N
    var = s_ref[1:2, :] / N - mu * mu
    a = g_ref[...] / jnp.sqrt(var + EPS)
    c = be_ref[...] - mu * a
    hb = jnp.maximum(t_ref[...] * a + c, 0.0)
    dinv = 1.0 / jnp.sqrt(deg_ref[...])
    o_ref[...] = jnp.dot(hb, w_ref[...],
                         preferred_element_type=jnp.float32) * dinv


def _bn_mm(t, s, g, be, w, deg):
    return pl.pallas_call(
        _bn_mm_body,
        grid=(GRID,),
        in_specs=[pl.BlockSpec((RB, H), lambda i: (i, 0)),
                  pl.BlockSpec((2, H), lambda i: (0, 0)),
                  pl.BlockSpec((1, H), lambda i: (0, 0)),
                  pl.BlockSpec((1, H), lambda i: (0, 0)),
                  pl.BlockSpec((H, H), lambda i: (0, 0)),
                  pl.BlockSpec((RB, 1), lambda i: (i, 0))],
        out_specs=pl.BlockSpec((RB, H), lambda i: (i, 0)),
        out_shape=jax.ShapeDtypeStruct((NROWS, H), jnp.float32),
    )(t, s, g, be, w, deg)


def _pool_body(t_ref, s_ref, g_ref, be_ref, batch_ref, wl_ref, bl_ref, o_ref,
               sums, counts):
    i = pl.program_id(0)

    @pl.when(i == 0)
    def _():
        sums[...] = jnp.zeros_like(sums)
        counts[...] = jnp.zeros_like(counts)

    mu = s_ref[0:1, :] / N
    var = s_ref[1:2, :] / N - mu * mu
    a = g_ref[...] / jnp.sqrt(var + EPS)
    c = be_ref[...] - mu * a
    h = jnp.maximum(t_ref[...] * a + c, 0.0)
    bid = batch_ref[0]
    gid = lax.broadcasted_iota(jnp.int32, (G, RB), 0)
    onehot = (bid == gid).astype(jnp.float32)
    sums[...] += jnp.dot(onehot, h, preferred_element_type=jnp.float32)
    counts[...] = counts[...] + jnp.broadcast_to(
        jnp.sum(onehot, axis=1, keepdims=True), (G, H))

    @pl.when(i == GRID - 1)
    def _():
        pooled = sums[...] / jnp.maximum(counts[...], 1.0)
        o_ref[...] = jnp.dot(pooled, wl_ref[...],
                             preferred_element_type=jnp.float32) + bl_ref[...]


def _pool(t, s, g, be, batch3, wlp, bl2):
    return pl.pallas_call(
        _pool_body,
        grid=(GRID,),
        in_specs=[pl.BlockSpec((RB, H), lambda i: (i, 0)),
                  pl.BlockSpec((2, H), lambda i: (0, 0)),
                  pl.BlockSpec((1, H), lambda i: (0, 0)),
                  pl.BlockSpec((1, H), lambda i: (0, 0)),
                  pl.BlockSpec((1, 1, RB), lambda i: (i, 0, 0)),
                  pl.BlockSpec((H, H), lambda i: (0, 0)),
                  pl.BlockSpec((1, 1), lambda i: (0, 0))],
        out_specs=pl.BlockSpec((G, H), lambda i: (0, 0)),
        out_shape=jax.ShapeDtypeStruct((G, H), jnp.float32),
        scratch_shapes=[pltpu.VMEM((G, H), jnp.float32),
                        pltpu.VMEM((G, H), jnp.float32)],
    )(t, s, g, be, batch3, wlp, bl2)


def kernel(x, edge_index, batch, W1, b1, g1, be1, W2, b2, g2, be2,
           W3, b3, g3, be3, Wl, bl):
    pad = jnp.full((NW, EPT - EPW), N, jnp.int32)
    src3 = jnp.concatenate([edge_index[0].reshape(NW, EPW), pad],
                           axis=1).reshape(NW, NCH, CK)
    dst3 = jnp.concatenate([edge_index[1].reshape(NW, EPW), pad],
                           axis=1).reshape(NW, NCH, CK)

    degp = _get_deg_call()(dst3)
    deg = _deg_combine(degp)[:N]

    batch3 = batch.reshape(GRID, 1, RB)
    wlp = jnp.pad(Wl, ((0, 0), (0, H - 1)))
    bl2 = bl.reshape(1, 1)
    b1r, b2r, b3r = b1.reshape(1, H), b2.reshape(1, H), b3.reshape(1, H)
    g1r, g2r, g3r = g1.reshape(1, H), g2.reshape(1, H), g3.reshape(1, H)
    be1r, be2r, be3r = be1.reshape(1, H), be2.reshape(1, H), be3.reshape(1, H)

    xs1 = _mm_scale(x, W1, deg)
    p1 = _get_edge_call()(xs1, src3, dst3)
    t1, s1 = _stats(p1, xs1, deg, b1r)

    xs2 = _bn_mm(t1, s1, g1r, be1r, W2, deg)
    p2 = _get_edge_call()(xs2, src3, dst3)
    t2, s2 = _stats(p2, xs2, deg, b2r)

    xs3 = _bn_mm(t2, s2, g2r, be2r, W3, deg)
    p3 = _get_edge_call()(xs3, src3, dst3)
    t3, s3 = _stats(p3, xs3, deg, b3r)

    pout = _pool(t3, s3, g3r, be3r, batch3, wlp, bl2)
    return pout[:, :1]


# speedup vs baseline: 3.5827x; 3.5827x over previous
import functools

import jax
import jax.numpy as jnp
from jax import lax
from jax.experimental import pallas as pl
from jax.experimental.pallas import tpu as pltpu
from jax.experimental.pallas import tpu_sc as plsc

N = 10000
E = 320000
H = 128
G = 64
EPS = 1e-5

NC, NS = 2, 16
NW = NC * NS
EPW = E // NW
CK = 80
NCH = 125
NSL = 5
SLC = NCH // NSL
NB = 3
EPT = NCH * CK
ROWS_PT = 640
DEG_EPT = E // NS
NPAD16 = 640

RB = 2000
GRID = N // RB


NROWS = NPAD16 * 16


def _deg_body(dst_hbm, out_hbm, didx, ones_rows, zbuf, shared):
    cid = lax.axis_index("c")
    sid = lax.axis_index("s")
    wid = sid * NC + cid
    ones16 = jnp.ones((16,), jnp.float32)
    zeros16 = jnp.zeros((16,), jnp.float32)

    def fill(r, c):
        ones_rows[r] = ones16
        zbuf[r] = zeros16
        return c
    lax.fori_loop(0, CK, fill, 0)
    for r in range(640 // CK):
        pltpu.sync_copy(zbuf, shared.at[pl.ds(sid * 640 + r * CK, CK)])
    plsc.subcore_barrier()

    pltpu.sync_copy(dst_hbm.at[wid], didx)

    def body(j, c):
        pltpu.sync_copy(ones_rows, shared.at[didx.at[j]], add=True)
        return c
    lax.fori_loop(0, NCH, body, 0)

    plsc.subcore_barrier()
    pltpu.sync_copy(shared.at[pl.ds(sid * 640, 640)],
                    out_hbm.at[cid, pl.ds(sid * 640, 640)])


@functools.cache
def _get_deg_call():
    return pl.kernel(
        _deg_body,
        out_type=jax.ShapeDtypeStruct((NC, NROWS, 16), jnp.float32),
        mesh=plsc.VectorSubcoreMesh(core_axis_name="c", subcore_axis_name="s",
                                    num_cores=NC, num_subcores=NS),
        scratch_types=[
            pltpu.VMEM((NCH, CK), jnp.int32),
            pltpu.VMEM((CK, 16), jnp.float32),
            pltpu.VMEM((CK, 16), jnp.float32),
            pltpu.VMEM_SHARED((NROWS, 16), jnp.float32),
        ],
    )


def _deg_combine_body(p_ref, o_ref):
    o_ref[...] = (jnp.sum(p_ref[0], axis=1, keepdims=True)
                  + jnp.sum(p_ref[1], axis=1, keepdims=True)) / 16.0 + 1.0


def _deg_combine(p):
    return pl.pallas_call(
        _deg_combine_body,
        out_shape=jax.ShapeDtypeStruct((NROWS, 1), jnp.float32),
    )(p)


def _edge_body(xs_hbm, src_hbm, dst4_hbm, out_hbm, sidx, didx, rows0, rows1,
               rows2, shared, gsem0, gsem1, gsem2, ssem0, ssem1, ssem2):
    cid = lax.axis_index("c")
    sid = lax.axis_index("s")
    wid = sid * NC + cid
    zeros16 = jnp.zeros((16,), jnp.float32)

    def zr(r, c):
        def zc(cc, c2):
            rows0[r, pl.ds(cc * 16, 16)] = zeros16
            return c2
        return lax.fori_loop(0, H // 16, zc, c)
    lax.fori_loop(0, CK, zr, 0)
    for r in range(ROWS_PT // CK):
        pltpu.sync_copy(rows0, shared.at[pl.ds(sid * ROWS_PT + r * CK, CK)])
    plsc.subcore_barrier()

    pltpu.sync_copy(src_hbm.at[pl.ds(wid * EPT, EPT)], sidx)

    rows = (rows0, rows1, rows2)
    gsems = (gsem0, gsem1, gsem2)
    ssems = (ssem0, ssem1, ssem2)

    def slab(h):
        pltpu.sync_copy(dst4_hbm.at[wid, h], didx)
        dg = [None] * SLC
        ds = [None] * SLC
        for i in range(SLC):
            if i >= NB:
                ds[i - NB].wait()
            dg[i] = pltpu.async_copy(
                xs_hbm.at[sidx.at[pl.ds((h * SLC + i) * CK, CK)]],
                rows[i % NB], gsems[i % NB])
            if i >= NB - 1:
                j = i - (NB - 1)
                dg[j].wait()
                ds[j] = pltpu.async_copy(rows[j % NB],
                                         shared.at[didx.at[j]],
                                         ssems[j % NB], add=True)
        for j in range(SLC - (NB - 1), SLC):
            dg[j].wait()
            ds[j] = pltpu.async_copy(rows[j % NB], shared.at[didx.at[j]],
                                     ssems[j % NB], add=True)
        for j in range(SLC - NB, SLC):
            ds[j].wait()

    def gbody(h, c):
        slab(h)
        return c
    lax.fori_loop(0, NSL, gbody, 0)

    plsc.subcore_barrier()
    pltpu.sync_copy(shared.at[pl.ds(sid * ROWS_PT, ROWS_PT)],
                    out_hbm.at[cid, pl.ds(sid * ROWS_PT, ROWS_PT)])


@functools.cache
def _get_edge_call():
    return pl.kernel(
        _edge_body,
        out_type=jax.ShapeDtypeStruct((NC, NROWS, H), jnp.float32),
        mesh=plsc.VectorSubcoreMesh(core_axis_name="c", subcore_axis_name="s",
                                    num_cores=NC, num_subcores=NS),
        scratch_types=[
            pltpu.VMEM((EPT,), jnp.int32),
            pltpu.VMEM((SLC, CK), jnp.int32),
            pltpu.VMEM((CK, H), jnp.float32),
            pltpu.VMEM((CK, H), jnp.float32),
            pltpu.VMEM((CK, H), jnp.float32),
            pltpu.VMEM_SHARED((NROWS, H), jnp.float32),
            pltpu.SemaphoreType.DMA,
            pltpu.SemaphoreType.DMA,
            pltpu.SemaphoreType.DMA,
            pltpu.SemaphoreType.DMA,
            pltpu.SemaphoreType.DMA,
            pltpu.SemaphoreType.DMA,
        ],
    )


def _mm_scale_body(x_ref, w_ref, deg_ref, o_ref):
    dinv = 1.0 / jnp.sqrt(deg_ref[...])
    o_ref[...] = jnp.dot(x_ref[...], w_ref[...],
                         preferred_element_type=jnp.float32) * dinv


def _mm_scale(x, w, deg):
    return pl.pallas_call(
        _mm_scale_body,
        grid=(GRID,),
        in_specs=[pl.BlockSpec((RB, H), lambda i: (i, 0)),
                  pl.BlockSpec((H, H), lambda i: (0, 0)),
                  pl.BlockSpec((RB, 1), lambda i: (i, 0))],
        out_specs=pl.BlockSpec((RB, H), lambda i: (i, 0)),
        out_shape=jax.ShapeDtypeStruct((NROWS, H), jnp.float32),
    )(x, w, deg)


def _stats_body(p_ref, xs_ref, deg_ref, b_ref, t_ref, s_ref):
    i = pl.program_id(0)
    dinv = 1.0 / jnp.sqrt(deg_ref[...])
    t = (p_ref[0] + p_ref[1] + xs_ref[...]) * dinv + b_ref[...]
    t_ref[...] = t

    @pl.when(i == 0)
    def _():
        s_ref[...] = jnp.zeros_like(s_ref)

    s_ref[0:1, :] += jnp.sum(t, axis=0, keepdims=True)
    s_ref[1:2, :] += jnp.sum(t * t, axis=0, keepdims=True)


def _stats(p, xs, deg, b):
    return pl.pallas_call(
        _stats_body,
        grid=(GRID,),
        in_specs=[pl.BlockSpec((NC, RB, H), lambda i: (0, i, 0)),
                  pl.BlockSpec((RB, H), lambda i: (i, 0)),
                  pl.BlockSpec((RB, 1), lambda i: (i, 0)),
                  pl.BlockSpec((1, H), lambda i: (0, 0))],
        out_specs=[pl.BlockSpec((RB, H), lambda i: (i, 0)),
                   pl.BlockSpec((2, H), lambda i: (0, 0))],
        out_shape=[jax.ShapeDtypeStruct((N, H), jnp.float32),
                   jax.ShapeDtypeStruct((2, H), jnp.float32)],
    )(p, xs, deg, b)


def _bn_mm_body(t_ref, s_ref, g_ref, be_ref, w_ref, deg_ref, o_ref):
    mu = s_ref[0:1, :] / N
    var = s_ref[1:2, :] / N - mu * mu
    a = g_ref[...] / jnp.sqrt(var + EPS)
    c = be_ref[...] - mu * a
    hb = jnp.maximum(t_ref[...] * a + c, 0.0)
    dinv = 1.0 / jnp.sqrt(deg_ref[...])
    o_ref[...] = jnp.dot(hb, w_ref[...],
                         preferred_element_type=jnp.float32) * dinv


def _bn_mm(t, s, g, be, w, deg):
    return pl.pallas_call(
        _bn_mm_body,
        grid=(GRID,),
        in_specs=[pl.BlockSpec((RB, H), lambda i: (i, 0)),
                  pl.BlockSpec((2, H), lambda i: (0, 0)),
                  pl.BlockSpec((1, H), lambda i: (0, 0)),
                  pl.BlockSpec((1, H), lambda i: (0, 0)),
                  pl.BlockSpec((H, H), lambda i: (0, 0)),
                  pl.BlockSpec((RB, 1), lambda i: (i, 0))],
        out_specs=pl.BlockSpec((RB, H), lambda i: (i, 0)),
        out_shape=jax.ShapeDtypeStruct((NROWS, H), jnp.float32),
    )(t, s, g, be, w, deg)


def _pool_body(t_ref, s_ref, g_ref, be_ref, batch_ref, wl_ref, bl_ref, o_ref,
               sums, counts):
    i = pl.program_id(0)

    @pl.when(i == 0)
    def _():
        sums[...] = jnp.zeros_like(sums)
        counts[...] = jnp.zeros_like(counts)

    mu = s_ref[0:1, :] / N
    var = s_ref[1:2, :] / N - mu * mu
    a = g_ref[...] / jnp.sqrt(var + EPS)
    c = be_ref[...] - mu * a
    h = jnp.maximum(t_ref[...] * a + c, 0.0)
    bid = batch_ref[0]
    gid = lax.broadcasted_iota(jnp.int32, (G, RB), 0)
    onehot = (bid == gid).astype(jnp.float32)
    sums[...] += jnp.dot(onehot, h, preferred_element_type=jnp.float32)
    counts[...] = counts[...] + jnp.broadcast_to(
        jnp.sum(onehot, axis=1, keepdims=True), (G, H))

    @pl.when(i == GRID - 1)
    def _():
        pooled = sums[...] / jnp.maximum(counts[...], 1.0)
        o_ref[...] = jnp.dot(pooled, wl_ref[...],
                             preferred_element_type=jnp.float32) + bl_ref[...]


def _pool(t, s, g, be, batch3, wlp, bl2):
    return pl.pallas_call(
        _pool_body,
        grid=(GRID,),
        in_specs=[pl.BlockSpec((RB, H), lambda i: (i, 0)),
                  pl.BlockSpec((2, H), lambda i: (0, 0)),
                  pl.BlockSpec((1, H), lambda i: (0, 0)),
                  pl.BlockSpec((1, H), lambda i: (0, 0)),
                  pl.BlockSpec((1, 1, RB), lambda i: (i, 0, 0)),
                  pl.BlockSpec((H, H), lambda i: (0, 0)),
                  pl.BlockSpec((1, 1), lambda i: (0, 0))],
        out_specs=pl.BlockSpec((G, H), lambda i: (0, 0)),
        out_shape=jax.ShapeDtypeStruct((G, H), jnp.float32),
        scratch_shapes=[pltpu.VMEM((G, H), jnp.float32),
                        pltpu.VMEM((G, H), jnp.float32)],
    )(t, s, g, be, batch3, wlp, bl2)


def kernel(x, edge_index, batch, W1, b1, g1, be1, W2, b2, g2, be2,
           W3, b3, g3, be3, Wl, bl):
    srcf = edge_index[0]
    dst3 = edge_index[1].reshape(NW, NCH, CK)
    dst4 = edge_index[1].reshape(NW, NSL, SLC, CK)

    degp = _get_deg_call()(dst3)
    deg = _deg_combine(degp)[:N]

    batch3 = batch.reshape(GRID, 1, RB)
    wlp = jnp.pad(Wl, ((0, 0), (0, H - 1)))
    bl2 = bl.reshape(1, 1)
    b1r, b2r, b3r = b1.reshape(1, H), b2.reshape(1, H), b3.reshape(1, H)
    g1r, g2r, g3r = g1.reshape(1, H), g2.reshape(1, H), g3.reshape(1, H)
    be1r, be2r, be3r = be1.reshape(1, H), be2.reshape(1, H), be3.reshape(1, H)

    xs1 = _mm_scale(x, W1, deg)
    p1 = _get_edge_call()(xs1, srcf, dst4)
    t1, s1 = _stats(p1, xs1, deg, b1r)

    xs2 = _bn_mm(t1, s1, g1r, be1r, W2, deg)
    p2 = _get_edge_call()(xs2, srcf, dst4)
    t2, s2 = _stats(p2, xs2, deg, b2r)

    xs3 = _bn_mm(t2, s2, g2r, be2r, W3, deg)
    p3 = _get_edge_call()(xs3, srcf, dst4)
    t3, s3 = _stats(p3, xs3, deg, b3r)

    pout = _pool(t3, s3, g3r, be3r, batch3, wlp, bl2)
    return pout[:, :1]


# speedup vs baseline: 3.6345x; 1.0145x over previous
import functools

import jax
import jax.numpy as jnp
from jax import lax
from jax.experimental import pallas as pl
from jax.experimental.pallas import tpu as pltpu
from jax.experimental.pallas import tpu_sc as plsc

N = 10000
E = 320000
H = 128
G = 64
EPS = 1e-5

NC, NS = 2, 16
NW = NC * NS
EPW = E // NW
CK = 80
NCH = 125
NSL = 5
SLC = NCH // NSL
NB = 3
EPT = NCH * CK
ROWS_PT = 640
DEG_EPT = E // NS
NPAD16 = 640

RB = 2000
GRID = N // RB


NROWS = NPAD16 * 16


def _deg_body(dst_hbm, out_hbm, didx, ones_rows, zbuf, shared, dsem):
    cid = lax.axis_index("c")
    sid = lax.axis_index("s")
    wid = sid * NC + cid
    ones16 = jnp.ones((16,), jnp.float32)
    zeros16 = jnp.zeros((16,), jnp.float32)

    def fill(r, c):
        ones_rows[r] = ones16
        zbuf[r] = zeros16
        return c
    lax.fori_loop(0, CK, fill, 0)
    for r in range(640 // CK):
        pltpu.sync_copy(zbuf, shared.at[pl.ds(sid * 640 + r * CK, CK)])
    plsc.subcore_barrier()

    pltpu.sync_copy(dst_hbm.at[wid], didx)

    def dbody(h, c):
        ds = [pltpu.async_copy(ones_rows, shared.at[didx.at[h * SLC + i]],
                               dsem, add=True)
              for i in range(SLC)]
        for d in ds:
            d.wait()
        return c
    lax.fori_loop(0, NSL, dbody, 0)

    plsc.subcore_barrier()
    pltpu.sync_copy(shared.at[pl.ds(sid * 640, 640)],
                    out_hbm.at[cid, pl.ds(sid * 640, 640)])


@functools.cache
def _get_deg_call():
    return pl.kernel(
        _deg_body,
        out_type=jax.ShapeDtypeStruct((NC, NROWS, 16), jnp.float32),
        mesh=plsc.VectorSubcoreMesh(core_axis_name="c", subcore_axis_name="s",
                                    num_cores=NC, num_subcores=NS),
        scratch_types=[
            pltpu.VMEM((NCH, CK), jnp.int32),
            pltpu.VMEM((CK, 16), jnp.float32),
            pltpu.VMEM((CK, 16), jnp.float32),
            pltpu.VMEM_SHARED((NROWS, 16), jnp.float32),
            pltpu.SemaphoreType.DMA,
        ],
    )


def _deg_combine_body(p_ref, o_ref):
    o_ref[...] = (jnp.sum(p_ref[0], axis=1, keepdims=True)
                  + jnp.sum(p_ref[1], axis=1, keepdims=True)) / 16.0 + 1.0


def _deg_combine(p):
    return pl.pallas_call(
        _deg_combine_body,
        out_shape=jax.ShapeDtypeStruct((NROWS, 1), jnp.float32),
    )(p)


def _edge_body(xs_hbm, src_hbm, dst4_hbm, out_hbm, sidx, didx, rows0, rows1,
               rows2, shared, gsem0, gsem1, gsem2, ssem0, ssem1, ssem2):
    cid = lax.axis_index("c")
    sid = lax.axis_index("s")
    wid = sid * NC + cid
    zeros16 = jnp.zeros((16,), jnp.float32)

    def zr(r, c):
        def zc(cc, c2):
            rows0[r, pl.ds(cc * 16, 16)] = zeros16
            return c2
        return lax.fori_loop(0, H // 16, zc, c)
    lax.fori_loop(0, CK, zr, 0)
    for r in range(ROWS_PT // CK):
        pltpu.sync_copy(rows0, shared.at[pl.ds(sid * ROWS_PT + r * CK, CK)])
    plsc.subcore_barrier()

    pltpu.sync_copy(src_hbm.at[pl.ds(wid * EPT, EPT)], sidx)

    rows = (rows0, rows1, rows2)
    gsems = (gsem0, gsem1, gsem2)
    ssems = (ssem0, ssem1, ssem2)

    def slab(h):
        pltpu.sync_copy(dst4_hbm.at[wid, h], didx)
        dg = [None] * SLC
        ds = [None] * SLC
        for i in range(SLC):
            if i >= NB:
                ds[i - NB].wait()
            dg[i] = pltpu.async_copy(
                xs_hbm.at[sidx.at[pl.ds((h * SLC + i) * CK, CK)]],
                rows[i % NB], gsems[i % NB])
            if i >= NB - 1:
                j = i - (NB - 1)
                dg[j].wait()
                ds[j] = pltpu.async_copy(rows[j % NB],
                                         shared.at[didx.at[j]],
                                         ssems[j % NB], add=True)
        for j in range(SLC - (NB - 1), SLC):
            dg[j].wait()
            ds[j] = pltpu.async_copy(rows[j % NB], shared.at[didx.at[j]],
                                     ssems[j % NB], add=True)
        for j in range(SLC - NB, SLC):
            ds[j].wait()

    def gbody(h, c):
        slab(h)
        return c
    lax.fori_loop(0, NSL, gbody, 0)

    plsc.subcore_barrier()
    pltpu.sync_copy(shared.at[pl.ds(sid * ROWS_PT, ROWS_PT)],
                    out_hbm.at[cid, pl.ds(sid * ROWS_PT, ROWS_PT)])


@functools.cache
def _get_edge_call():
    return pl.kernel(
        _edge_body,
        out_type=jax.ShapeDtypeStruct((NC, NROWS, H), jnp.float32),
        mesh=plsc.VectorSubcoreMesh(core_axis_name="c", subcore_axis_name="s",
                                    num_cores=NC, num_subcores=NS),
        scratch_types=[
            pltpu.VMEM((EPT,), jnp.int32),
            pltpu.VMEM((SLC, CK), jnp.int32),
            pltpu.VMEM((CK, H), jnp.float32),
            pltpu.VMEM((CK, H), jnp.float32),
            pltpu.VMEM((CK, H), jnp.float32),
            pltpu.VMEM_SHARED((NROWS, H), jnp.float32),
            pltpu.SemaphoreType.DMA,
            pltpu.SemaphoreType.DMA,
            pltpu.SemaphoreType.DMA,
            pltpu.SemaphoreType.DMA,
            pltpu.SemaphoreType.DMA,
            pltpu.SemaphoreType.DMA,
        ],
    )


def _mm_scale_body(x_ref, w_ref, deg_ref, o_ref):
    dinv = 1.0 / jnp.sqrt(deg_ref[...])
    o_ref[...] = jnp.dot(x_ref[...], w_ref[...],
                         preferred_element_type=jnp.float32) * dinv


def _mm_scale(x, w, deg):
    return pl.pallas_call(
        _mm_scale_body,
        grid=(GRID,),
        in_specs=[pl.BlockSpec((RB, H), lambda i: (i, 0)),
                  pl.BlockSpec((H, H), lambda i: (0, 0)),
                  pl.BlockSpec((RB, 1), lambda i: (i, 0))],
        out_specs=pl.BlockSpec((RB, H), lambda i: (i, 0)),
        out_shape=jax.ShapeDtypeStruct((NROWS, H), jnp.float32),
    )(x, w, deg)


def _stats_body(p_ref, xs_ref, deg_ref, b_ref, t_ref, s_ref):
    i = pl.program_id(0)
    dinv = 1.0 / jnp.sqrt(deg_ref[...])
    t = (p_ref[0] + p_ref[1] + xs_ref[...]) * dinv + b_ref[...]
    t_ref[...] = t

    @pl.when(i == 0)
    def _():
        s_ref[...] = jnp.zeros_like(s_ref)

    s_ref[0:1, :] += jnp.sum(t, axis=0, keepdims=True)
    s_ref[1:2, :] += jnp.sum(t * t, axis=0, keepdims=True)


def _stats(p, xs, deg, b):
    return pl.pallas_call(
        _stats_body,
        grid=(GRID,),
        in_specs=[pl.BlockSpec((NC, RB, H), lambda i: (0, i, 0)),
                  pl.BlockSpec((RB, H), lambda i: (i, 0)),
                  pl.BlockSpec((RB, 1), lambda i: (i, 0)),
                  pl.BlockSpec((1, H), lambda i: (0, 0))],
        out_specs=[pl.BlockSpec((RB, H), lambda i: (i, 0)),
                   pl.BlockSpec((2, H), lambda i: (0, 0))],
        out_shape=[jax.ShapeDtypeStruct((N, H), jnp.float32),
                   jax.ShapeDtypeStruct((2, H), jnp.float32)],
    )(p, xs, deg, b)


def _bn_mm_body(t_ref, s_ref, g_ref, be_ref, w_ref, deg_ref, o_ref):
    mu = s_ref[0:1, :] / N
    var = s_ref[1:2, :] / N - mu * mu
    a = g_ref[...] / jnp.sqrt(var + EPS)
    c = be_ref[...] - mu * a
    hb = jnp.maximum(t_ref[...] * a + c, 0.0)
    dinv = 1.0 / jnp.sqrt(deg_ref[...])
    o_ref[...] = jnp.dot(hb, w_ref[...],
                         preferred_element_type=jnp.float32) * dinv


def _bn_mm(t, s, g, be, w, deg):
    return pl.pallas_call(
        _bn_mm_body,
        grid=(GRID,),
        in_specs=[pl.BlockSpec((RB, H), lambda i: (i, 0)),
                  pl.BlockSpec((2, H), lambda i: (0, 0)),
                  pl.BlockSpec((1, H), lambda i: (0, 0)),
                  pl.BlockSpec((1, H), lambda i: (0, 0)),
                  pl.BlockSpec((H, H), lambda i: (0, 0)),
                  pl.BlockSpec((RB, 1), lambda i: (i, 0))],
        out_specs=pl.BlockSpec((RB, H), lambda i: (i, 0)),
        out_shape=jax.ShapeDtypeStruct((NROWS, H), jnp.float32),
    )(t, s, g, be, w, deg)


def _pool_body(t_ref, s_ref, g_ref, be_ref, batch_ref, wl_ref, bl_ref, o_ref,
               sums, counts):
    i = pl.program_id(0)

    @pl.when(i == 0)
    def _():
        sums[...] = jnp.zeros_like(sums)
        counts[...] = jnp.zeros_like(counts)

    mu = s_ref[0:1, :] / N
    var = s_ref[1:2, :] / N - mu * mu
    a = g_ref[...] / jnp.sqrt(var + EPS)
    c = be_ref[...] - mu * a
    h = jnp.maximum(t_ref[...] * a + c, 0.0)
    bid = batch_ref[0]
    gid = lax.broadcasted_iota(jnp.int32, (G, RB), 0)
    onehot = (bid == gid).astype(jnp.float32)
    sums[...] += jnp.dot(onehot, h, preferred_element_type=jnp.float32)
    counts[...] = counts[...] + jnp.broadcast_to(
        jnp.sum(onehot, axis=1, keepdims=True), (G, H))

    @pl.when(i == GRID - 1)
    def _():
        pooled = sums[...] / jnp.maximum(counts[...], 1.0)
        o_ref[...] = jnp.dot(pooled, wl_ref[...],
                             preferred_element_type=jnp.float32) + bl_ref[...]


def _pool(t, s, g, be, batch3, wlp, bl2):
    return pl.pallas_call(
        _pool_body,
        grid=(GRID,),
        in_specs=[pl.BlockSpec((RB, H), lambda i: (i, 0)),
                  pl.BlockSpec((2, H), lambda i: (0, 0)),
                  pl.BlockSpec((1, H), lambda i: (0, 0)),
                  pl.BlockSpec((1, H), lambda i: (0, 0)),
                  pl.BlockSpec((1, 1, RB), lambda i: (i, 0, 0)),
                  pl.BlockSpec((H, H), lambda i: (0, 0)),
                  pl.BlockSpec((1, 1), lambda i: (0, 0))],
        out_specs=pl.BlockSpec((G, H), lambda i: (0, 0)),
        out_shape=jax.ShapeDtypeStruct((G, H), jnp.float32),
        scratch_shapes=[pltpu.VMEM((G, H), jnp.float32),
                        pltpu.VMEM((G, H), jnp.float32)],
    )(t, s, g, be, batch3, wlp, bl2)


def kernel(x, edge_index, batch, W1, b1, g1, be1, W2, b2, g2, be2,
           W3, b3, g3, be3, Wl, bl):
    srcf = edge_index[0]
    dst3 = edge_index[1].reshape(NW, NCH, CK)
    dst4 = edge_index[1].reshape(NW, NSL, SLC, CK)

    degp = _get_deg_call()(dst3)
    deg = _deg_combine(degp)[:N]

    batch3 = batch.reshape(GRID, 1, RB)
    wlp = jnp.pad(Wl, ((0, 0), (0, H - 1)))
    bl2 = bl.reshape(1, 1)
    b1r, b2r, b3r = b1.reshape(1, H), b2.reshape(1, H), b3.reshape(1, H)
    g1r, g2r, g3r = g1.reshape(1, H), g2.reshape(1, H), g3.reshape(1, H)
    be1r, be2r, be3r = be1.reshape(1, H), be2.reshape(1, H), be3.reshape(1, H)

    xs1 = _mm_scale(x, W1, deg)
    p1 = _get_edge_call()(xs1, srcf, dst4)
    t1, s1 = _stats(p1, xs1, deg, b1r)

    xs2 = _bn_mm(t1, s1, g1r, be1r, W2, deg)
    p2 = _get_edge_call()(xs2, srcf, dst4)
    t2, s2 = _stats(p2, xs2, deg, b2r)

    xs3 = _bn_mm(t2, s2, g2r, be2r, W3, deg)
    p3 = _get_edge_call()(xs3, srcf, dst4)
    t3, s3 = _stats(p3, xs3, deg, b3r)

    pout = _pool(t3, s3, g3r, be3r, batch3, wlp, bl2)
    return pout[:, :1]
